# Initial kernel scaffold; baseline (speedup 1.0000x reference)
import jax
import jax.numpy as jnp
from jax.experimental import pallas as pl


def kernel(x_user, x_item, edge_index_u2i, edge_index_i2u, params):
    raise NotImplementedError("write your pallas kernel here")


# speedup vs baseline: 3.5806x; 3.5806x over previous
import functools

import jax
import jax.numpy as jnp
from jax import lax
from jax.experimental import pallas as pl
from jax.experimental.pallas import tpu as pltpu
from jax.experimental.pallas import tpu_sc as plsc

N = 50000
D_IN = 128
H = 64
NUM_LAYERS = 2
E = 800000

NS = 16
NC = 2
NPAD = 50176
HALF = NPAD // 2
RPT = HALF // NS
TRASH = 64
ACC_ROWS = HALF + TRASH

TB = 98
NTB = RPT // TB
CH = 128
EPT_CHUNKS = 392
EPT = CH * EPT_CHUNKS
EP = EPT * NS

BLK = 1568
GRID = NPAD // BLK


def _sc_segsum_pair(h_user, h_item, src_u, dst_i, src_i, dst_u):
    mesh = plsc.VectorSubcoreMesh(core_axis_name="c", subcore_axis_name="s")

    @functools.partial(
        pl.kernel, mesh=mesh,
        compiler_params=pltpu.CompilerParams(use_tc_tiling_on_sc=False),
        out_type=[jax.ShapeDtypeStruct((NPAD, H), jnp.float32),
                  jax.ShapeDtypeStruct((NPAD, H), jnp.float32)],
        scratch_types=[
            pltpu.VMEM_SHARED((ACC_ROWS, H), jnp.float32),
            pltpu.VMEM((2, CH), jnp.int32),
            pltpu.VMEM((2, CH), jnp.int32),
            pltpu.VMEM((CH,), jnp.int32),
            pltpu.VMEM((2, CH, H), jnp.float32),
            pltpu.VMEM((TB, H), jnp.float32),
            pltpu.SemaphoreType.DMA((2,)),
        ],
    )
    def k(hu, hi, su, di, si, du, mi_out, mu_out,
          acc, sidx, didx, lidx, rows, tbuf, sem):
        c = lax.axis_index("c")
        s = lax.axis_index("s")
        base_dst = c * HALF
        ebase = s * EPT
        zero16 = jnp.zeros((16,), jnp.float32)

        for (table, src, dst, mout) in ((hu, su, di, mi_out),
                                        (hi, si, du, mu_out)):
            def zrow(r, carry):
                for k4 in range(H // 16):
                    tbuf[r, pl.ds(k4 * 16, 16)] = zero16
                return carry
            lax.fori_loop(0, TB, zrow, 0)

            def zcopy(hh, carry):
                pltpu.sync_copy(tbuf, acc.at[pl.ds(s * RPT + hh * TB, TB)])
                return carry
            lax.fori_loop(0, NTB, zcopy, 0)
            plsc.subcore_barrier()

            pltpu.sync_copy(src.at[pl.ds(ebase, CH)], sidx.at[0])
            pltpu.sync_copy(dst.at[pl.ds(ebase, CH)], didx.at[0])
            pltpu.make_async_copy(table.at[sidx.at[0]], rows.at[0],
                                  sem.at[0]).start()

            def chunk_pair(g2, carry):
                for b in range(2):
                    g = g2 * 2 + b
                    nxt = b ^ 1

                    @pl.when(g + 1 < EPT_CHUNKS)
                    def _prefetch():
                        off = ebase + (g + 1) * CH
                        pltpu.sync_copy(src.at[pl.ds(off, CH)], sidx.at[nxt])
                        pltpu.sync_copy(dst.at[pl.ds(off, CH)], didx.at[nxt])
                        pltpu.make_async_copy(table.at[sidx.at[nxt]],
                                              rows.at[nxt], sem.at[nxt]).start()

                    pltpu.make_async_copy(table.at[sidx.at[b]], rows.at[b],
                                          sem.at[b]).wait()
                    for j in range(CH // 16):
                        dv = didx[b, pl.ds(j * 16, 16)]
                        lv = dv - base_dst
                        oob = (lv < 0) | (lv >= HALF)
                        tv = HALF + jnp.bitwise_and(dv, TRASH - 1)
                        lidx[pl.ds(j * 16, 16)] = jnp.where(oob, tv, lv)
                    pltpu.sync_copy(rows.at[b], acc.at[lidx], add=True)
                return carry

            lax.fori_loop(0, EPT_CHUNKS // 2, chunk_pair, 0)
            plsc.subcore_barrier()

            def wcopy(hh, carry):
                r0 = s * RPT + hh * TB
                pltpu.sync_copy(acc.at[pl.ds(r0, TB)], tbuf)
                pltpu.sync_copy(tbuf, mout.at[pl.ds(base_dst + r0, TB)])
                return carry
            lax.fori_loop(0, NTB, wcopy, 0)
            plsc.subcore_barrier()

    return k(h_user, h_item, src_u, dst_i, src_i, dst_u)


def _row_spec(d):
    return pl.BlockSpec((BLK, d), lambda i: (i, 0))


def _full_spec(shape):
    nd = len(shape)
    return pl.BlockSpec(shape, lambda i, _nd=nd: (0,) * _nd)


def _proj_body(xu, xi, wu, bu, wi, bi, hu, hi):
    hu[...] = jnp.dot(xu[...], wu[...], preferred_element_type=jnp.float32) + bu[...]
    hi[...] = jnp.dot(xi[...], wi[...], preferred_element_type=jnp.float32) + bi[...]


def _tc_proj(xu, xi, wu, bu, wi, bi):
    return pl.pallas_call(
        _proj_body,
        grid=(GRID,),
        in_specs=[_row_spec(D_IN), _row_spec(D_IN),
                  _full_spec((D_IN, H)), _full_spec((1, H)),
                  _full_spec((D_IN, H)), _full_spec((1, H))],
        out_specs=[_row_spec(H), _row_spec(H)],
        out_shape=[jax.ShapeDtypeStruct((NPAD, H), jnp.float32),
                   jax.ShapeDtypeStruct((NPAD, H), jnp.float32)],
    )(xu, xi, wu, bu, wi, bi)


def _gin_body(hu, mu, w1u, b1u, w2u, b2u, hi, mi, w1i, b1i, w2i, b2i, ou, oi):
    for (h, m, w1, b1, w2, b2, o) in ((hu, mu, w1u, b1u, w2u, b2u, ou),
                                      (hi, mi, w1i, b1i, w2i, b2i, oi)):
        z = h[...] + m[...]
        t = jnp.maximum(jnp.dot(z, w1[...], preferred_element_type=jnp.float32)
                        + b1[...], 0.0)
        t = jnp.dot(t, w2[...], preferred_element_type=jnp.float32) + b2[...]
        o[...] = jnp.maximum(t, 0.0)


def _tc_gin(hu, mu, wu, hi, mi, wi):
    (w1u, b1u), (w2u, b2u) = wu
    (w1i, b1i), (w2i, b2i) = wi
    return pl.pallas_call(
        _gin_body,
        grid=(GRID,),
        in_specs=[_row_spec(H), _row_spec(H),
                  _full_spec((H, H)), _full_spec((1, H)),
                  _full_spec((H, H)), _full_spec((1, H)),
                  _row_spec(H), _row_spec(H),
                  _full_spec((H, H)), _full_spec((1, H)),
                  _full_spec((H, H)), _full_spec((1, H))],
        out_specs=[_row_spec(H), _row_spec(H)],
        out_shape=[jax.ShapeDtypeStruct((NPAD, H), jnp.float32),
                   jax.ShapeDtypeStruct((NPAD, H), jnp.float32)],
    )(hu, mu, w1u, b1u, w2u, b2u, hi, mi, w1i, b1i, w2i, b2i)


def _layer_norm(x, g, b):
    mu = jnp.mean(x, axis=-1, keepdims=True)
    v = jnp.var(x, axis=-1, keepdims=True)
    return (x - mu) / jnp.sqrt(v + 1e-5) * g + b


def _final_body(*refs):
    (hu1, hu2, hi1, hi2) = refs[0:4]
    wu = refs[4:16]
    wi = refs[16:28]
    emb_u, emb_i, out_u, out_i = refs[28:32]
    for (h1, h2, w, emb, out) in ((hu1, hu2, wu, emb_u, out_u),
                                  (hi1, hi2, wi, emb_i, out_i)):
        (jkw, jkb, m1w, m1b, g1, be1, m2w, m2b, g2, be2, m3w, m3b) = w[:12]
        cat = jnp.concatenate([h1[...], h2[...]], axis=-1)
        e = jnp.dot(cat, jkw[...], preferred_element_type=jnp.float32) + jkb[...]
        emb[...] = e
        t = jnp.dot(e, m1w[...], preferred_element_type=jnp.float32) + m1b[...]
        t = jnp.maximum(_layer_norm(t, g1[...], be1[...]), 0.0)
        t = jnp.dot(t, m2w[...], preferred_element_type=jnp.float32) + m2b[...]
        t = jnp.maximum(_layer_norm(t, g2[...], be2[...]), 0.0)
        out[...] = jnp.dot(t, m3w[...], preferred_element_type=jnp.float32) + m3b[...]


def _tc_final(hu1, hu2, hi1, hi2, wu, wi):
    shapes = [(2 * H, H), (1, H),
              (H, 2 * H), (1, 2 * H),
              (1, 2 * H), (1, 2 * H),
              (2 * H, 2 * H), (1, 2 * H),
              (1, 2 * H), (1, 2 * H),
              (2 * H, 32), (1, 32)]
    w_specs = [_full_spec(s) for s in shapes]
    return pl.pallas_call(
        _final_body,
        grid=(GRID,),
        in_specs=[_row_spec(H)] * 4 + w_specs + w_specs,
        out_specs=[_row_spec(H), _row_spec(H), _row_spec(32), _row_spec(32)],
        out_shape=[jax.ShapeDtypeStruct((NPAD, H), jnp.float32),
                   jax.ShapeDtypeStruct((NPAD, H), jnp.float32),
                   jax.ShapeDtypeStruct((NPAD, 32), jnp.float32),
                   jax.ShapeDtypeStruct((NPAD, 32), jnp.float32)],
    )(hu1, hu2, hi1, hi2, *wu, *wi)


def _rb(b):
    return b.reshape(1, -1)


def kernel(x_user, x_item, edge_index_u2i, edge_index_i2u, params):
    p = params
    xu = jnp.pad(x_user, ((0, NPAD - N), (0, 0)))
    xi = jnp.pad(x_item, ((0, NPAD - N), (0, 0)))

    pad_n = EP - E
    pad_src = jnp.full((pad_n,), NPAD - 1, jnp.int32)
    pad_dst = N + (jnp.arange(pad_n, dtype=jnp.int32) % (NPAD - N))

    def prep(ei):
        return (jnp.concatenate([ei[0], pad_src]),
                jnp.concatenate([ei[1], pad_dst]))

    su, di = prep(edge_index_u2i)
    si, du = prep(edge_index_i2u)

    hu, hi = _tc_proj(xu, xi,
                      p['proj_user'][0], _rb(p['proj_user'][1]),
                      p['proj_item'][0], _rb(p['proj_item'][1]))

    hs_u, hs_i = [], []
    for l in range(NUM_LAYERS):
        mi, mu = _sc_segsum_pair(hu, hi, su, di, si, du)
        wu = ((p['gin%d_i2u_1' % l][0], _rb(p['gin%d_i2u_1' % l][1])),
              (p['gin%d_i2u_2' % l][0], _rb(p['gin%d_i2u_2' % l][1])))
        wi = ((p['gin%d_u2i_1' % l][0], _rb(p['gin%d_u2i_1' % l][1])),
              (p['gin%d_u2i_2' % l][0], _rb(p['gin%d_u2i_2' % l][1])))
        hu, hi = _tc_gin(hu, mu, wu, hi, mi, wi)
        hs_u.append(hu)
        hs_i.append(hi)

    def head_w(t):
        return [p['jk_' + t][0], _rb(p['jk_' + t][1]),
                p['mlp_%s_1' % t][0], _rb(p['mlp_%s_1' % t][1]),
                _rb(p['mlp_%s_ln1' % t][0]), _rb(p['mlp_%s_ln1' % t][1]),
                p['mlp_%s_2' % t][0], _rb(p['mlp_%s_2' % t][1]),
                _rb(p['mlp_%s_ln2' % t][0]), _rb(p['mlp_%s_ln2' % t][1]),
                p['mlp_%s_3' % t][0], _rb(p['mlp_%s_3' % t][1])]

    emb_u, emb_i, out_u, out_i = _tc_final(hs_u[0], hs_u[1], hs_i[0], hs_i[1],
                                           head_w('user'), head_w('item'))
    return (emb_u[:N], emb_i[:N], out_u[:N], out_i[:N])


# speedup vs baseline: 4.9707x; 1.3882x over previous
import functools

import jax
import jax.numpy as jnp
from jax import lax
from jax.experimental import pallas as pl
from jax.experimental.pallas import tpu as pltpu
from jax.experimental.pallas import tpu_sc as plsc

N = 50000
D_IN = 128
H = 64
NUM_LAYERS = 2
E = 800000

NS = 16
NC = 2
NPAD = 50176
HALF = NPAD // 2
RPT = HALF // NS
TRASH = 64
ACC_ROWS = HALF + TRASH

TB = 56
NTB = RPT // TB
CH = 128
BCH = 14
NBLK = 28
EPT = CH * BCH * NBLK
EP = EPT * NS

BLK = 1568
GRID = NPAD // BLK


def _sc_segsum_pair(h_user, h_item, src_u, dst_i, src_i, dst_u):
    mesh = plsc.VectorSubcoreMesh(core_axis_name="c", subcore_axis_name="s")

    @functools.partial(
        pl.kernel, mesh=mesh,
        compiler_params=pltpu.CompilerParams(use_tc_tiling_on_sc=False),
        out_type=[jax.ShapeDtypeStruct((NPAD, H), jnp.float32),
                  jax.ShapeDtypeStruct((NPAD, H), jnp.float32)],
        scratch_types=[
            pltpu.VMEM_SHARED((ACC_ROWS, H), jnp.float32),
            pltpu.VMEM((2, BCH, CH), jnp.int32),
            pltpu.VMEM((2, BCH, CH), jnp.int32),
            pltpu.VMEM((2, CH), jnp.int32),
            pltpu.VMEM((2, CH, H), jnp.float32),
            pltpu.VMEM((TB, H), jnp.float32),
            pltpu.SemaphoreType.DMA((2,)),
            pltpu.SemaphoreType.DMA((2,)),
            pltpu.SemaphoreType.DMA((2,)),
        ],
    )
    def k(hu, hi, su, di, si, du, mi_out, mu_out,
          acc, sidx, didx, lidx, rows, tbuf, isem, gsem, ssem):
        c = lax.axis_index("c")
        s = lax.axis_index("s")
        base_dst = c * HALF
        zero16 = jnp.zeros((16,), jnp.float32)

        for (table, src, dst, mout) in ((hu, su, di, mi_out),
                                        (hi, si, du, mu_out)):
            def zrow(r, carry):
                for k4 in range(H // 16):
                    tbuf[r, pl.ds(k4 * 16, 16)] = zero16
                return carry
            lax.fori_loop(0, TB, zrow, 0)

            def zcopy(hh, carry):
                pltpu.sync_copy(tbuf, acc.at[pl.ds(s * RPT + hh * TB, TB)])
                return carry
            lax.fori_loop(0, NTB, zcopy, 0)
            plsc.subcore_barrier()

            def idx_start(blk, ibuf):
                pltpu.make_async_copy(src.at[s, blk], sidx.at[ibuf],
                                      isem.at[ibuf]).start()
                pltpu.make_async_copy(dst.at[s, blk], didx.at[ibuf],
                                      isem.at[ibuf]).start()

            def idx_wait(ibuf):
                pltpu.make_async_copy(src.at[s, 0], sidx.at[ibuf],
                                      isem.at[ibuf]).wait()
                pltpu.make_async_copy(dst.at[s, 0], didx.at[ibuf],
                                      isem.at[ibuf]).wait()

            def g_start(ibuf, pos, rb):
                pltpu.make_async_copy(table.at[sidx.at[ibuf, pos]],
                                      rows.at[rb], gsem.at[rb]).start()

            def g_wait(rb):
                pltpu.make_async_copy(table.at[sidx.at[0, 0]],
                                      rows.at[rb], gsem.at[rb]).wait()

            def s_start(rb):
                pltpu.async_copy(rows.at[rb], acc.at[lidx.at[rb]],
                                 ssem.at[rb], add=True)

            def s_wait(rb):
                pltpu.make_async_copy(rows.at[rb], acc.at[lidx.at[rb]],
                                      ssem.at[rb]).wait()

            def remap(ibuf, pos, rb):
                for j16 in range(CH // 16):
                    dv = didx[ibuf, pos, pl.ds(j16 * 16, 16)]
                    lv = dv - base_dst
                    oob = (lv < 0) | (lv >= HALF)
                    tv = HALF + jnp.bitwise_and(dv, TRASH - 1)
                    lidx[rb, pl.ds(j16 * 16, 16)] = jnp.where(oob, tv, lv)

            idx_start(0, 0)
            idx_wait(0)
            idx_start(1, 1)
            g_start(0, 0, 0)

            def outer(blk, carry):
                ib = jnp.bitwise_and(blk, 1)
                nib = 1 - ib

                def pair(j2, carry2):
                    @pl.when(blk + j2 > 0)
                    def _():
                        s_wait(1)
                    g_start(ib, 2 * j2 + 1, 1)
                    g_wait(0)
                    remap(ib, 2 * j2, 0)
                    s_start(0)
                    s_wait(0)
                    last = j2 == BCH // 2 - 1

                    @pl.when(~last)
                    def _():
                        g_start(ib, 2 * j2 + 2, 0)

                    @pl.when(last & (blk + 1 < NBLK))
                    def _():
                        idx_wait(nib)
                        g_start(nib, 0, 0)
                    g_wait(1)

                    @pl.when(last & (blk + 2 < NBLK))
                    def _():
                        idx_start(blk + 2, ib)
                    remap(ib, 2 * j2 + 1, 1)
                    s_start(1)
                    return carry2

                return lax.fori_loop(0, BCH // 2, pair, carry)

            lax.fori_loop(0, NBLK, outer, 0)
            s_wait(1)
            plsc.subcore_barrier()

            def wcopy(hh, carry):
                r0 = s * RPT + hh * TB
                pltpu.sync_copy(acc.at[pl.ds(r0, TB)], tbuf)
                pltpu.sync_copy(tbuf, mout.at[pl.ds(base_dst + r0, TB)])
                return carry
            lax.fori_loop(0, NTB, wcopy, 0)
            plsc.subcore_barrier()

    return k(h_user, h_item, src_u, dst_i, src_i, dst_u)


def _row_spec(d):
    return pl.BlockSpec((BLK, d), lambda i: (i, 0))


def _full_spec(shape):
    nd = len(shape)
    return pl.BlockSpec(shape, lambda i, _nd=nd: (0,) * _nd)


def _proj_body(xu, xi, wu, bu, wi, bi, hu, hi):
    hu[...] = jnp.dot(xu[...], wu[...], preferred_element_type=jnp.float32) + bu[...]
    hi[...] = jnp.dot(xi[...], wi[...], preferred_element_type=jnp.float32) + bi[...]


def _tc_proj(xu, xi, wu, bu, wi, bi):
    return pl.pallas_call(
        _proj_body,
        grid=(GRID,),
        in_specs=[_row_spec(D_IN), _row_spec(D_IN),
                  _full_spec((D_IN, H)), _full_spec((1, H)),
                  _full_spec((D_IN, H)), _full_spec((1, H))],
        out_specs=[_row_spec(H), _row_spec(H)],
        out_shape=[jax.ShapeDtypeStruct((NPAD, H), jnp.float32),
                   jax.ShapeDtypeStruct((NPAD, H), jnp.float32)],
    )(xu, xi, wu, bu, wi, bi)


def _gin_body(hu, mu, w1u, b1u, w2u, b2u, hi, mi, w1i, b1i, w2i, b2i, ou, oi):
    for (h, m, w1, b1, w2, b2, o) in ((hu, mu, w1u, b1u, w2u, b2u, ou),
                                      (hi, mi, w1i, b1i, w2i, b2i, oi)):
        z = h[...] + m[...]
        t = jnp.maximum(jnp.dot(z, w1[...], preferred_element_type=jnp.float32)
                        + b1[...], 0.0)
        t = jnp.dot(t, w2[...], preferred_element_type=jnp.float32) + b2[...]
        o[...] = jnp.maximum(t, 0.0)


def _tc_gin(hu, mu, wu, hi, mi, wi):
    (w1u, b1u), (w2u, b2u) = wu
    (w1i, b1i), (w2i, b2i) = wi
    return pl.pallas_call(
        _gin_body,
        grid=(GRID,),
        in_specs=[_row_spec(H), _row_spec(H),
                  _full_spec((H, H)), _full_spec((1, H)),
                  _full_spec((H, H)), _full_spec((1, H)),
                  _row_spec(H), _row_spec(H),
                  _full_spec((H, H)), _full_spec((1, H)),
                  _full_spec((H, H)), _full_spec((1, H))],
        out_specs=[_row_spec(H), _row_spec(H)],
        out_shape=[jax.ShapeDtypeStruct((NPAD, H), jnp.float32),
                   jax.ShapeDtypeStruct((NPAD, H), jnp.float32)],
    )(hu, mu, w1u, b1u, w2u, b2u, hi, mi, w1i, b1i, w2i, b2i)


def _layer_norm(x, g, b):
    mu = jnp.mean(x, axis=-1, keepdims=True)
    v = jnp.var(x, axis=-1, keepdims=True)
    return (x - mu) / jnp.sqrt(v + 1e-5) * g + b


def _final_body(*refs):
    (hu1, hu2, hi1, hi2) = refs[0:4]
    wu = refs[4:16]
    wi = refs[16:28]
    emb_u, emb_i, out_u, out_i = refs[28:32]
    for (h1, h2, w, emb, out) in ((hu1, hu2, wu, emb_u, out_u),
                                  (hi1, hi2, wi, emb_i, out_i)):
        (jkw, jkb, m1w, m1b, g1, be1, m2w, m2b, g2, be2, m3w, m3b) = w[:12]
        cat = jnp.concatenate([h1[...], h2[...]], axis=-1)
        e = jnp.dot(cat, jkw[...], preferred_element_type=jnp.float32) + jkb[...]
        emb[...] = e
        t = jnp.dot(e, m1w[...], preferred_element_type=jnp.float32) + m1b[...]
        t = jnp.maximum(_layer_norm(t, g1[...], be1[...]), 0.0)
        t = jnp.dot(t, m2w[...], preferred_element_type=jnp.float32) + m2b[...]
        t = jnp.maximum(_layer_norm(t, g2[...], be2[...]), 0.0)
        out[...] = jnp.dot(t, m3w[...], preferred_element_type=jnp.float32) + m3b[...]


def _tc_final(hu1, hu2, hi1, hi2, wu, wi):
    shapes = [(2 * H, H), (1, H),
              (H, 2 * H), (1, 2 * H),
              (1, 2 * H), (1, 2 * H),
              (2 * H, 2 * H), (1, 2 * H),
              (1, 2 * H), (1, 2 * H),
              (2 * H, 32), (1, 32)]
    w_specs = [_full_spec(s) for s in shapes]
    return pl.pallas_call(
        _final_body,
        grid=(GRID,),
        in_specs=[_row_spec(H)] * 4 + w_specs + w_specs,
        out_specs=[_row_spec(H), _row_spec(H), _row_spec(32), _row_spec(32)],
        out_shape=[jax.ShapeDtypeStruct((NPAD, H), jnp.float32),
                   jax.ShapeDtypeStruct((NPAD, H), jnp.float32),
                   jax.ShapeDtypeStruct((NPAD, 32), jnp.float32),
                   jax.ShapeDtypeStruct((NPAD, 32), jnp.float32)],
    )(hu1, hu2, hi1, hi2, *wu, *wi)


def _rb(b):
    return b.reshape(1, -1)


def kernel(x_user, x_item, edge_index_u2i, edge_index_i2u, params):
    p = params
    xu = jnp.pad(x_user, ((0, NPAD - N), (0, 0)))
    xi = jnp.pad(x_item, ((0, NPAD - N), (0, 0)))

    pad_n = EP - E
    pad_src = jnp.full((pad_n,), NPAD - 1, jnp.int32)
    pad_dst = N + (jnp.arange(pad_n, dtype=jnp.int32) % (NPAD - N))

    def prep(ei):
        s4 = jnp.concatenate([ei[0], pad_src]).reshape(NS, NBLK, BCH, CH)
        d4 = jnp.concatenate([ei[1], pad_dst]).reshape(NS, NBLK, BCH, CH)
        return s4, d4

    su, di = prep(edge_index_u2i)
    si, du = prep(edge_index_i2u)

    hu, hi = _tc_proj(xu, xi,
                      p['proj_user'][0], _rb(p['proj_user'][1]),
                      p['proj_item'][0], _rb(p['proj_item'][1]))

    hs_u, hs_i = [], []
    for l in range(NUM_LAYERS):
        mi, mu = _sc_segsum_pair(hu, hi, su, di, si, du)
        wu = ((p['gin%d_i2u_1' % l][0], _rb(p['gin%d_i2u_1' % l][1])),
              (p['gin%d_i2u_2' % l][0], _rb(p['gin%d_i2u_2' % l][1])))
        wi = ((p['gin%d_u2i_1' % l][0], _rb(p['gin%d_u2i_1' % l][1])),
              (p['gin%d_u2i_2' % l][0], _rb(p['gin%d_u2i_2' % l][1])))
        hu, hi = _tc_gin(hu, mu, wu, hi, mi, wi)
        hs_u.append(hu)
        hs_i.append(hi)

    def head_w(t):
        return [p['jk_' + t][0], _rb(p['jk_' + t][1]),
                p['mlp_%s_1' % t][0], _rb(p['mlp_%s_1' % t][1]),
                _rb(p['mlp_%s_ln1' % t][0]), _rb(p['mlp_%s_ln1' % t][1]),
                p['mlp_%s_2' % t][0], _rb(p['mlp_%s_2' % t][1]),
                _rb(p['mlp_%s_ln2' % t][0]), _rb(p['mlp_%s_ln2' % t][1]),
                p['mlp_%s_3' % t][0], _rb(p['mlp_%s_3' % t][1])]

    emb_u, emb_i, out_u, out_i = _tc_final(hs_u[0], hs_u[1], hs_i[0], hs_i[1],
                                           head_w('user'), head_w('item'))
    return (emb_u[:N], emb_i[:N], out_u[:N], out_i[:N])


# speedup vs baseline: 4.9742x; 1.0007x over previous
import functools

import jax
import jax.numpy as jnp
from jax import lax
from jax.experimental import pallas as pl
from jax.experimental.pallas import tpu as pltpu
from jax.experimental.pallas import tpu_sc as plsc

N = 50000
D_IN = 128
H = 64
NUM_LAYERS = 2
E = 800000

NS = 16
NC = 2
NPAD = 50176
HALF = NPAD // 2
RPT = HALF // NS
TRASH = 512
ACC_ROWS = HALF + TRASH

TB = 56
NTB = RPT // TB
CH = 128
BCH = 14
NBLK = 28
EPT = CH * BCH * NBLK
EP = EPT * NS

BLK = 1568
GRID = NPAD // BLK


def _sc_segsum_pair(h_user, h_item, src_u, dst_i, src_i, dst_u):
    mesh = plsc.VectorSubcoreMesh(core_axis_name="c", subcore_axis_name="s")

    @functools.partial(
        pl.kernel, mesh=mesh,
        compiler_params=pltpu.CompilerParams(use_tc_tiling_on_sc=False),
        out_type=[jax.ShapeDtypeStruct((NPAD, H), jnp.float32),
                  jax.ShapeDtypeStruct((NPAD, H), jnp.float32)],
        scratch_types=[
            pltpu.VMEM_SHARED((ACC_ROWS, H), jnp.float32),
            pltpu.VMEM((2, BCH, CH), jnp.int32),
            pltpu.VMEM((2, BCH, CH), jnp.int32),
            pltpu.VMEM((2, CH), jnp.int32),
            pltpu.VMEM((2, CH, H), jnp.float32),
            pltpu.VMEM((TB, H), jnp.float32),
            pltpu.SemaphoreType.DMA((2,)),
            pltpu.SemaphoreType.DMA((2,)),
            pltpu.SemaphoreType.DMA((2,)),
        ],
    )
    def k(hu, hi, su, di, si, du, mi_out, mu_out,
          acc, sidx, didx, lidx, rows, tbuf, isem, gsem, ssem):
        c = lax.axis_index("c")
        s = lax.axis_index("s")
        base_dst = c * HALF
        zero16 = jnp.zeros((16,), jnp.float32)

        for (table, src, dst, mout) in ((hu, su, di, mi_out),
                                        (hi, si, du, mu_out)):
            def zrow(r, carry):
                for k4 in range(H // 16):
                    tbuf[r, pl.ds(k4 * 16, 16)] = zero16
                return carry
            lax.fori_loop(0, TB, zrow, 0)

            def zcopy(hh, carry):
                pltpu.sync_copy(tbuf, acc.at[pl.ds(s * RPT + hh * TB, TB)])
                return carry
            lax.fori_loop(0, NTB, zcopy, 0)
            plsc.subcore_barrier()

            def idx_start(blk, ibuf):
                pltpu.make_async_copy(src.at[s, blk], sidx.at[ibuf],
                                      isem.at[ibuf]).start()
                pltpu.make_async_copy(dst.at[s, blk], didx.at[ibuf],
                                      isem.at[ibuf]).start()

            def idx_wait(ibuf):
                pltpu.make_async_copy(src.at[s, 0], sidx.at[ibuf],
                                      isem.at[ibuf]).wait()
                pltpu.make_async_copy(dst.at[s, 0], didx.at[ibuf],
                                      isem.at[ibuf]).wait()

            def g_start(ibuf, pos, rb):
                pltpu.make_async_copy(table.at[sidx.at[ibuf, pos]],
                                      rows.at[rb], gsem.at[rb]).start()

            def g_wait(rb):
                pltpu.make_async_copy(table.at[sidx.at[0, 0]],
                                      rows.at[rb], gsem.at[rb]).wait()

            def s_start(rb):
                pltpu.async_copy(rows.at[rb], acc.at[lidx.at[rb]],
                                 ssem.at[rb], add=True)

            def s_wait(rb):
                pltpu.make_async_copy(rows.at[rb], acc.at[lidx.at[rb]],
                                      ssem.at[rb]).wait()

            def remap(ibuf, pos, rb):
                for j16 in range(CH // 16):
                    dv = didx[ibuf, pos, pl.ds(j16 * 16, 16)]
                    lv = dv - base_dst
                    oob = (lv < 0) | (lv >= HALF)
                    tv = HALF + jnp.bitwise_and(dv, TRASH - 1)
                    lidx[rb, pl.ds(j16 * 16, 16)] = jnp.where(oob, tv, lv)

            idx_start(0, 0)
            idx_wait(0)
            idx_start(1, 1)
            g_start(0, 0, 0)

            def outer(blk, carry):
                ib = jnp.bitwise_and(blk, 1)
                nib = 1 - ib

                def pair(j2, carry2):
                    @pl.when(blk + j2 > 0)
                    def _():
                        s_wait(1)
                    g_start(ib, 2 * j2 + 1, 1)
                    g_wait(0)
                    remap(ib, 2 * j2, 0)
                    s_start(0)
                    s_wait(0)
                    last = j2 == BCH // 2 - 1

                    @pl.when(~last)
                    def _():
                        g_start(ib, 2 * j2 + 2, 0)

                    @pl.when(last & (blk + 1 < NBLK))
                    def _():
                        idx_wait(nib)
                        g_start(nib, 0, 0)
                    g_wait(1)

                    @pl.when(last & (blk + 2 < NBLK))
                    def _():
                        idx_start(blk + 2, ib)
                    remap(ib, 2 * j2 + 1, 1)
                    s_start(1)
                    return carry2

                return lax.fori_loop(0, BCH // 2, pair, carry)

            lax.fori_loop(0, NBLK, outer, 0)
            s_wait(1)
            plsc.subcore_barrier()

            def wcopy(hh, carry):
                r0 = s * RPT + hh * TB
                pltpu.sync_copy(acc.at[pl.ds(r0, TB)], tbuf)
                pltpu.sync_copy(tbuf, mout.at[pl.ds(base_dst + r0, TB)])
                return carry
            lax.fori_loop(0, NTB, wcopy, 0)
            plsc.subcore_barrier()

    return k(h_user, h_item, src_u, dst_i, src_i, dst_u)


def _row_spec(d):
    return pl.BlockSpec((BLK, d), lambda i: (i, 0))


def _full_spec(shape):
    nd = len(shape)
    return pl.BlockSpec(shape, lambda i, _nd=nd: (0,) * _nd)


def _proj_body(xu, xi, wu, bu, wi, bi, hu, hi):
    hu[...] = jnp.dot(xu[...], wu[...], preferred_element_type=jnp.float32) + bu[...]
    hi[...] = jnp.dot(xi[...], wi[...], preferred_element_type=jnp.float32) + bi[...]


def _tc_proj(xu, xi, wu, bu, wi, bi):
    return pl.pallas_call(
        _proj_body,
        grid=(GRID,),
        in_specs=[_row_spec(D_IN), _row_spec(D_IN),
                  _full_spec((D_IN, H)), _full_spec((1, H)),
                  _full_spec((D_IN, H)), _full_spec((1, H))],
        out_specs=[_row_spec(H), _row_spec(H)],
        out_shape=[jax.ShapeDtypeStruct((NPAD, H), jnp.float32),
                   jax.ShapeDtypeStruct((NPAD, H), jnp.float32)],
    )(xu, xi, wu, bu, wi, bi)


def _gin_body(hu, mu, w1u, b1u, w2u, b2u, hi, mi, w1i, b1i, w2i, b2i, ou, oi):
    for (h, m, w1, b1, w2, b2, o) in ((hu, mu, w1u, b1u, w2u, b2u, ou),
                                      (hi, mi, w1i, b1i, w2i, b2i, oi)):
        z = h[...] + m[...]
        t = jnp.maximum(jnp.dot(z, w1[...], preferred_element_type=jnp.float32)
                        + b1[...], 0.0)
        t = jnp.dot(t, w2[...], preferred_element_type=jnp.float32) + b2[...]
        o[...] = jnp.maximum(t, 0.0)


def _tc_gin(hu, mu, wu, hi, mi, wi):
    (w1u, b1u), (w2u, b2u) = wu
    (w1i, b1i), (w2i, b2i) = wi
    return pl.pallas_call(
        _gin_body,
        grid=(GRID,),
        in_specs=[_row_spec(H), _row_spec(H),
                  _full_spec((H, H)), _full_spec((1, H)),
                  _full_spec((H, H)), _full_spec((1, H)),
                  _row_spec(H), _row_spec(H),
                  _full_spec((H, H)), _full_spec((1, H)),
                  _full_spec((H, H)), _full_spec((1, H))],
        out_specs=[_row_spec(H), _row_spec(H)],
        out_shape=[jax.ShapeDtypeStruct((NPAD, H), jnp.float32),
                   jax.ShapeDtypeStruct((NPAD, H), jnp.float32)],
    )(hu, mu, w1u, b1u, w2u, b2u, hi, mi, w1i, b1i, w2i, b2i)


def _layer_norm(x, g, b):
    mu = jnp.mean(x, axis=-1, keepdims=True)
    v = jnp.var(x, axis=-1, keepdims=True)
    return (x - mu) / jnp.sqrt(v + 1e-5) * g + b


def _final_body(*refs):
    (hu1, hu2, hi1, hi2) = refs[0:4]
    wu = refs[4:16]
    wi = refs[16:28]
    emb_u, emb_i, out_u, out_i = refs[28:32]
    for (h1, h2, w, emb, out) in ((hu1, hu2, wu, emb_u, out_u),
                                  (hi1, hi2, wi, emb_i, out_i)):
        (jkw, jkb, m1w, m1b, g1, be1, m2w, m2b, g2, be2, m3w, m3b) = w[:12]
        cat = jnp.concatenate([h1[...], h2[...]], axis=-1)
        e = jnp.dot(cat, jkw[...], preferred_element_type=jnp.float32) + jkb[...]
        emb[...] = e
        t = jnp.dot(e, m1w[...], preferred_element_type=jnp.float32) + m1b[...]
        t = jnp.maximum(_layer_norm(t, g1[...], be1[...]), 0.0)
        t = jnp.dot(t, m2w[...], preferred_element_type=jnp.float32) + m2b[...]
        t = jnp.maximum(_layer_norm(t, g2[...], be2[...]), 0.0)
        out[...] = jnp.dot(t, m3w[...], preferred_element_type=jnp.float32) + m3b[...]


def _tc_final(hu1, hu2, hi1, hi2, wu, wi):
    shapes = [(2 * H, H), (1, H),
              (H, 2 * H), (1, 2 * H),
              (1, 2 * H), (1, 2 * H),
              (2 * H, 2 * H), (1, 2 * H),
              (1, 2 * H), (1, 2 * H),
              (2 * H, 32), (1, 32)]
    w_specs = [_full_spec(s) for s in shapes]
    return pl.pallas_call(
        _final_body,
        grid=(GRID,),
        in_specs=[_row_spec(H)] * 4 + w_specs + w_specs,
        out_specs=[_row_spec(H), _row_spec(H), _row_spec(32), _row_spec(32)],
        out_shape=[jax.ShapeDtypeStruct((NPAD, H), jnp.float32),
                   jax.ShapeDtypeStruct((NPAD, H), jnp.float32),
                   jax.ShapeDtypeStruct((NPAD, 32), jnp.float32),
                   jax.ShapeDtypeStruct((NPAD, 32), jnp.float32)],
    )(hu1, hu2, hi1, hi2, *wu, *wi)


def _rb(b):
    return b.reshape(1, -1)


def kernel(x_user, x_item, edge_index_u2i, edge_index_i2u, params):
    p = params
    xu = jnp.pad(x_user, ((0, NPAD - N), (0, 0)))
    xi = jnp.pad(x_item, ((0, NPAD - N), (0, 0)))

    pad_n = EP - E
    pad_src = jnp.full((pad_n,), NPAD - 1, jnp.int32)
    pad_dst = N + (jnp.arange(pad_n, dtype=jnp.int32) % (NPAD - N))

    def prep(ei):
        s4 = jnp.concatenate([ei[0], pad_src]).reshape(NS, NBLK, BCH, CH)
        d4 = jnp.concatenate([ei[1], pad_dst]).reshape(NS, NBLK, BCH, CH)
        return s4, d4

    su, di = prep(edge_index_u2i)
    si, du = prep(edge_index_i2u)

    hu, hi = _tc_proj(xu, xi,
                      p['proj_user'][0], _rb(p['proj_user'][1]),
                      p['proj_item'][0], _rb(p['proj_item'][1]))

    hs_u, hs_i = [], []
    for l in range(NUM_LAYERS):
        mi, mu = _sc_segsum_pair(hu, hi, su, di, si, du)
        wu = ((p['gin%d_i2u_1' % l][0], _rb(p['gin%d_i2u_1' % l][1])),
              (p['gin%d_i2u_2' % l][0], _rb(p['gin%d_i2u_2' % l][1])))
        wi = ((p['gin%d_u2i_1' % l][0], _rb(p['gin%d_u2i_1' % l][1])),
              (p['gin%d_u2i_2' % l][0], _rb(p['gin%d_u2i_2' % l][1])))
        hu, hi = _tc_gin(hu, mu, wu, hi, mi, wi)
        hs_u.append(hu)
        hs_i.append(hi)

    def head_w(t):
        return [p['jk_' + t][0], _rb(p['jk_' + t][1]),
                p['mlp_%s_1' % t][0], _rb(p['mlp_%s_1' % t][1]),
                _rb(p['mlp_%s_ln1' % t][0]), _rb(p['mlp_%s_ln1' % t][1]),
                p['mlp_%s_2' % t][0], _rb(p['mlp_%s_2' % t][1]),
                _rb(p['mlp_%s_ln2' % t][0]), _rb(p['mlp_%s_ln2' % t][1]),
                p['mlp_%s_3' % t][0], _rb(p['mlp_%s_3' % t][1])]

    emb_u, emb_i, out_u, out_i = _tc_final(hs_u[0], hs_u[1], hs_i[0], hs_i[1],
                                           head_w('user'), head_w('item'))
    return (emb_u[:N], emb_i[:N], out_u[:N], out_i[:N])
